# baseline (device time: 36731 ns/iter reference)
import jax
import jax.numpy as jnp
from jax import lax
from jax.experimental import pallas as pl
from jax.experimental.pallas import tpu as pltpu

M = 2048
HALF = 1024
XHALF = 512
D = 1024
CHUNK_ROWS = (32,) * 16
assert sum(CHUNK_ROWS) == XHALF
K = len(CHUNK_ROWS)
CHUNK_OFF = tuple(sum(CHUNK_ROWS[:c]) for c in range(K))
CRMAX = max(CHUNK_ROWS)


def kernel(partial, gamma):
    gamma2d = gamma.reshape(1, D)

    def body(p_ref, g_ref, out_ref, ybuf, xbuf, ysend, yrecv, xsend, xrecv,
             sem_x):
        my_x = lax.axis_index("x")
        my_y = lax.axis_index("y")
        peer_y = (my_x, 1 - my_y)
        peer_x = (1 - my_x, my_y)

        barrier_sem = pltpu.get_barrier_semaphore()
        pl.semaphore_signal(
            barrier_sem, inc=1, device_id=peer_y,
            device_id_type=pl.DeviceIdType.MESH,
        )
        pl.semaphore_signal(
            sem_x, inc=1, device_id=peer_x,
            device_id_type=pl.DeviceIdType.MESH,
        )
        pl.semaphore_wait(barrier_sem, 1)

        ysrc0 = (1 - my_y) * HALF + my_x * XHALF
        y_rdmas = []
        for c in range(K):
            r = pltpu.make_async_remote_copy(
                src_ref=p_ref.at[0, pl.ds(ysrc0 + CHUNK_OFF[c], CHUNK_ROWS[c]), :],
                dst_ref=ybuf.at[c, pl.ds(0, CHUNK_ROWS[c])],
                send_sem=ysend.at[c],
                recv_sem=yrecv.at[c],
                device_id=peer_y,
                device_id_type=pl.DeviceIdType.MESH,
            )
            r.start()
            y_rdmas.append(r)

        mine0 = my_y * HALF
        direct0 = my_x * XHALF
        fwd0 = (1 - my_x) * XHALF

        def rmsnorm_store(out_rows, local_rows, buf, c):
            cr = CHUNK_ROWS[c]
            y = p_ref[0, pl.ds(local_rows, cr), :] + buf[c, pl.ds(0, cr), :]
            ms = jnp.sum(y * y, axis=-1, keepdims=True) * (1.0 / D)
            out_ref[pl.ds(out_rows, cr), :] = y * lax.rsqrt(ms + 1e-6) * g_ref[...]

        x_rdmas = []
        for c in range(K):
            y_rdmas[c].wait_recv()
            if c == 0:
                pl.semaphore_wait(sem_x, 1)
            r = pltpu.make_async_remote_copy(
                src_ref=ybuf.at[c, pl.ds(0, CHUNK_ROWS[c])],
                dst_ref=xbuf.at[c, pl.ds(0, CHUNK_ROWS[c])],
                send_sem=xsend.at[c],
                recv_sem=xrecv.at[c],
                device_id=peer_x,
                device_id_type=pl.DeviceIdType.MESH,
            )
            r.start()
            x_rdmas.append(r)
            rmsnorm_store(direct0 + CHUNK_OFF[c], mine0 + direct0 + CHUNK_OFF[c],
                          ybuf, c)

        for c in range(K):
            x_rdmas[c].wait_recv()
            rmsnorm_store(fwd0 + CHUNK_OFF[c], mine0 + fwd0 + CHUNK_OFF[c],
                          xbuf, c)

        for c in range(K):
            y_rdmas[c].wait_send()
            x_rdmas[c].wait_send()

    return pl.pallas_call(
        body,
        out_shape=jax.ShapeDtypeStruct((HALF, D), jnp.float32),
        in_specs=[
            pl.BlockSpec(memory_space=pltpu.VMEM),
            pl.BlockSpec(memory_space=pltpu.VMEM),
        ],
        out_specs=pl.BlockSpec(memory_space=pltpu.VMEM),
        scratch_shapes=[
            pltpu.VMEM((K, CRMAX, D), jnp.float32),
            pltpu.VMEM((K, CRMAX, D), jnp.float32),
            pltpu.SemaphoreType.DMA((K,)),
            pltpu.SemaphoreType.DMA((K,)),
            pltpu.SemaphoreType.DMA((K,)),
            pltpu.SemaphoreType.DMA((K,)),
            pltpu.SemaphoreType.REGULAR,
        ],
        compiler_params=pltpu.CompilerParams(collective_id=0),
    )(partial, gamma2d)
